# Initial kernel scaffold; baseline (speedup 1.0000x reference)
import functools

import jax
import jax.numpy as jnp
from jax import lax
from jax.experimental import pallas as pl
from jax.experimental.pallas import tpu as pltpu
from jax.experimental.pallas import tpu_sc as plsc

NC = 2
NS = 16
NW = NC * NS
CHUNK = 80
HD = 16

_EPS = 1e-5


def _mesh():
    return plsc.VectorSubcoreMesh(
        core_axis_name="c", subcore_axis_name="s", num_cores=NC,
        num_subcores=NS)


def _deg_kernel(n_pad, cpt):
    rpt = n_pad // NS

    @functools.partial(
        pl.kernel,
        out_type=jax.ShapeDtypeStruct((NC, n_pad, HD), jnp.float32),
        mesh=_mesh(),
        scratch_types=[
            pltpu.VMEM((cpt, CHUNK), jnp.int32),
            pltpu.VMEM((CHUNK, HD), jnp.float32),
            pltpu.VMEM((rpt, HD), jnp.float32),
            pltpu.VMEM_SHARED((n_pad, HD), jnp.float32),
        ],
    )
    def deg_k(dst_hbm, out_hbm, dstv, ones_v, z_v, acc):
        c = lax.axis_index("c")
        s = lax.axis_index("s")
        wid = c * NS + s
        for r in range(CHUNK):
            ones_v[r, :] = jnp.ones((HD,), jnp.float32)
        for r in range(rpt):
            z_v[r, :] = jnp.zeros((HD,), jnp.float32)
        pltpu.sync_copy(z_v, acc.at[pl.ds(s * rpt, rpt)])
        plsc.subcore_barrier()
        pltpu.sync_copy(dst_hbm.at[wid], dstv)

        def body(j, carry):
            pltpu.sync_copy(ones_v, acc.at[dstv.at[j]], add=True)
            return carry

        lax.fori_loop(0, cpt, body, 0)
        plsc.subcore_barrier()
        pltpu.sync_copy(acc.at[pl.ds(s * rpt, rpt)],
                        out_hbm.at[c, pl.ds(s * rpt, rpt)])

    return deg_k


def _agg_kernel(n, n_pad, cpt, d):
    rpt = n_pad // NS
    zr = 64

    @functools.partial(
        pl.kernel,
        out_type=jax.ShapeDtypeStruct((NC, n_pad, d), jnp.float32),
        mesh=_mesh(),
        scratch_types=[
            pltpu.VMEM((cpt, CHUNK), jnp.int32),
            pltpu.VMEM((cpt, CHUNK), jnp.int32),
            pltpu.VMEM((CHUNK, d), jnp.float32),
            pltpu.VMEM((zr, d), jnp.float32),
            pltpu.VMEM_SHARED((n_pad, d), jnp.float32),
        ],
    )
    def agg_k(y_hbm, src_hbm, dst_hbm, out_hbm, srcv, dstv, rows, z_v, acc):
        c = lax.axis_index("c")
        s = lax.axis_index("s")
        wid = c * NS + s
        for r in range(zr):
            for k in range(d // 16):
                z_v[r, pl.ds(16 * k, 16)] = jnp.zeros((16,), jnp.float32)
        for b in range(rpt // zr):
            pltpu.sync_copy(z_v, acc.at[pl.ds(s * rpt + b * zr, zr)])
        pltpu.sync_copy(src_hbm.at[wid], srcv)
        pltpu.sync_copy(dst_hbm.at[wid], dstv)
        plsc.subcore_barrier()

        def body(j, carry):
            pltpu.sync_copy(y_hbm.at[srcv.at[j]], rows)
            pltpu.sync_copy(rows, acc.at[dstv.at[j]], add=True)
            return carry

        lax.fori_loop(0, cpt, body, 0)
        plsc.subcore_barrier()
        pltpu.sync_copy(acc.at[pl.ds(s * rpt, rpt)],
                        out_hbm.at[c, pl.ds(s * rpt, rpt)])

    return agg_k


def _tc_first(n, d):
    def body(x_ref, w_ref, degp_ref, y_ref, dinv_ref):
        deg = degp_ref[0, 0:n, 0:1] + degp_ref[1, 0:n, 0:1] + 1.0
        dinv = lax.rsqrt(deg)
        xw = jnp.dot(x_ref[...], w_ref[...],
                     preferred_element_type=jnp.float32)
        y_ref[...] = xw * dinv
        dinv_ref[...] = dinv

    return pl.pallas_call(
        body,
        out_shape=[
            jax.ShapeDtypeStruct((n, d), jnp.float32),
            jax.ShapeDtypeStruct((n, 1), jnp.float32),
        ],
    )


def _tc_mid(n, d):
    def body(aggp_ref, y_ref, dinv_ref, b_ref, g_ref, be_ref, a_ref, w2_ref,
             y2_ref):
        dinv = dinv_ref[...]
        agg = aggp_ref[0, 0:n, :] + aggp_ref[1, 0:n, :] + y_ref[...]
        h = dinv * agg + b_ref[...]
        mean = jnp.mean(h, axis=0, keepdims=True)
        cen = h - mean
        var = jnp.mean(cen * cen, axis=0, keepdims=True)
        hn = g_ref[...] * cen * lax.rsqrt(var + _EPS) + be_ref[...]
        act = jnp.where(hn >= 0, hn, a_ref[...] * hn)
        y2_ref[...] = jnp.dot(act, w2_ref[...],
                              preferred_element_type=jnp.float32) * dinv

    return pl.pallas_call(
        body,
        out_shape=jax.ShapeDtypeStruct((n, d), jnp.float32),
    )


def _tc_last(n, d):
    def body(aggp_ref, y_ref, dinv_ref, b_ref, g_ref, be_ref, out_ref):
        agg = aggp_ref[0, 0:n, :] + aggp_ref[1, 0:n, :] + y_ref[...]
        h = dinv_ref[...] * agg + b_ref[...]
        mean = jnp.mean(h, axis=0, keepdims=True)
        cen = h - mean
        var = jnp.mean(cen * cen, axis=0, keepdims=True)
        out_ref[...] = g_ref[...] * cen * lax.rsqrt(var + _EPS) + be_ref[...]

    return pl.pallas_call(
        body,
        out_shape=jax.ShapeDtypeStruct((n, d), jnp.float32),
    )


def kernel(x, edge_index, W1, b1, gamma1, beta1, alpha, W2, b2, gamma2,
           beta2):
    n, d_in = x.shape
    d_h = W1.shape[1]
    d_out = W2.shape[1]
    e = edge_index.shape[1]
    cpt = e // (NW * CHUNK)
    assert cpt * NW * CHUNK == e, "edge count must tile evenly"
    n_pad = ((n + NW * 32 - 1) // (NW * 32)) * (NW * 32)

    src3 = edge_index[0].reshape(NW, cpt, CHUNK)
    dst3 = edge_index[1].reshape(NW, cpt, CHUNK)

    b1r = b1.reshape(1, d_h)
    g1r = gamma1.reshape(1, d_h)
    be1r = beta1.reshape(1, d_h)
    ar = alpha.reshape(1, 1)
    b2r = b2.reshape(1, d_out)
    g2r = gamma2.reshape(1, d_out)
    be2r = beta2.reshape(1, d_out)

    degp = _deg_kernel(n_pad, cpt)(dst3)
    y1, dinv = _tc_first(n, d_h)(x, W1, degp)
    agg1 = _agg_kernel(n, n_pad, cpt, d_h)(y1, src3, dst3)
    y2 = _tc_mid(n, d_h)(agg1, y1, dinv, b1r, g1r, be1r, ar, W2)
    agg2 = _agg_kernel(n, n_pad, cpt, d_out)(y2, src3, dst3)
    out = _tc_last(n, d_out)(agg2, y2, dinv, b2r, g2r, be2r)
    return out


# speedup vs baseline: 9.1291x; 9.1291x over previous
import functools

import jax
import jax.numpy as jnp
from jax import lax
from jax.experimental import pallas as pl
from jax.experimental.pallas import tpu as pltpu
from jax.experimental.pallas import tpu_sc as plsc

NC = 2
NS = 16
NW = NC * NS
CHUNK = 128
HD = 16

_EPS = 1e-5


def _mesh():
    return plsc.VectorSubcoreMesh(
        core_axis_name="c", subcore_axis_name="s", num_cores=NC,
        num_subcores=NS)


BC = 8


def _deg_kernel(n_pad, cpt):
    rpt = n_pad // NS
    nb = cpt // BC

    @functools.partial(
        pl.kernel,
        out_type=jax.ShapeDtypeStruct((NC, n_pad, HD), jnp.float32),
        mesh=_mesh(),
        scratch_types=[
            pltpu.VMEM((BC, CHUNK), jnp.int32),
            pltpu.VMEM((CHUNK, HD), jnp.float32),
            pltpu.VMEM((CHUNK, HD), jnp.float32),
            pltpu.VMEM_SHARED((n_pad, HD), jnp.float32),
        ],
    )
    def deg_k(dst_hbm, out_hbm, dstv, ones_v, z_v, acc):
        c = lax.axis_index("c")
        s = lax.axis_index("s")
        wid = c * NS + s
        for r in range(CHUNK):
            ones_v[r, :] = jnp.ones((HD,), jnp.float32)
            z_v[r, :] = jnp.zeros((HD,), jnp.float32)
        for b in range(rpt // CHUNK):
            pltpu.sync_copy(z_v, acc.at[pl.ds(s * rpt + b * CHUNK, CHUNK)])
        plsc.subcore_barrier()

        def body(j, carry):
            pltpu.sync_copy(ones_v, acc.at[dstv.at[j]], add=True)
            return carry

        for blk in range(nb):
            pltpu.sync_copy(dst_hbm.at[wid, pl.ds(blk * BC, BC)], dstv)
            lax.fori_loop(0, BC, body, 0)
        plsc.subcore_barrier()
        pltpu.sync_copy(acc.at[pl.ds(s * rpt, rpt)],
                        out_hbm.at[c, pl.ds(s * rpt, rpt)])

    return deg_k


def _agg_kernel(n, n_pad, cpt, d):
    rpt = n_pad // NS
    nb = cpt // BC

    @functools.partial(
        pl.kernel,
        out_type=jax.ShapeDtypeStruct((NC, n_pad, d), jnp.float32),
        mesh=_mesh(),
        scratch_types=[
            pltpu.VMEM((BC, CHUNK), jnp.int32),
            pltpu.VMEM((BC, CHUNK), jnp.int32),
            pltpu.VMEM((CHUNK, d), jnp.float32),
            pltpu.VMEM_SHARED((n_pad, d), jnp.float32),
        ],
    )
    def agg_k(y_hbm, src_hbm, dst_hbm, out_hbm, srcv, dstv, rows, acc):
        c = lax.axis_index("c")
        s = lax.axis_index("s")
        wid = c * NS + s
        for r in range(CHUNK):
            for k in range(d // 16):
                rows[r, pl.ds(16 * k, 16)] = jnp.zeros((16,), jnp.float32)
        for b in range(rpt // CHUNK):
            pltpu.sync_copy(rows, acc.at[pl.ds(s * rpt + b * CHUNK, CHUNK)])
        plsc.subcore_barrier()

        def body(j, carry):
            pltpu.sync_copy(y_hbm.at[srcv.at[j]], rows)
            pltpu.sync_copy(rows, acc.at[dstv.at[j]], add=True)
            return carry

        for blk in range(nb):
            pltpu.sync_copy(src_hbm.at[wid, pl.ds(blk * BC, BC)], srcv)
            pltpu.sync_copy(dst_hbm.at[wid, pl.ds(blk * BC, BC)], dstv)
            lax.fori_loop(0, BC, body, 0)
        plsc.subcore_barrier()
        pltpu.sync_copy(acc.at[pl.ds(s * rpt, rpt)],
                        out_hbm.at[c, pl.ds(s * rpt, rpt)])

    return agg_k


def _tc_first(n, d):
    def body(x_ref, w_ref, degp_ref, y_ref, dinv_ref):
        deg = degp_ref[0, 0:n, 0:1] + degp_ref[1, 0:n, 0:1] + 1.0
        dinv = lax.rsqrt(deg)
        xw = jnp.dot(x_ref[...], w_ref[...],
                     preferred_element_type=jnp.float32)
        y_ref[...] = xw * dinv
        dinv_ref[...] = dinv

    return pl.pallas_call(
        body,
        out_shape=[
            jax.ShapeDtypeStruct((n, d), jnp.float32),
            jax.ShapeDtypeStruct((n, 1), jnp.float32),
        ],
    )


def _tc_mid(n, d):
    def body(aggp_ref, y_ref, dinv_ref, b_ref, g_ref, be_ref, a_ref, w2_ref,
             y2_ref):
        dinv = dinv_ref[...]
        agg = aggp_ref[0, 0:n, :] + aggp_ref[1, 0:n, :] + y_ref[...]
        h = dinv * agg + b_ref[...]
        mean = jnp.mean(h, axis=0, keepdims=True)
        cen = h - mean
        var = jnp.mean(cen * cen, axis=0, keepdims=True)
        hn = g_ref[...] * cen * lax.rsqrt(var + _EPS) + be_ref[...]
        act = jnp.where(hn >= 0, hn, a_ref[...] * hn)
        y2_ref[...] = jnp.dot(act, w2_ref[...],
                              preferred_element_type=jnp.float32) * dinv

    return pl.pallas_call(
        body,
        out_shape=jax.ShapeDtypeStruct((n, d), jnp.float32),
    )


def _tc_last(n, d):
    def body(aggp_ref, y_ref, dinv_ref, b_ref, g_ref, be_ref, out_ref):
        agg = aggp_ref[0, 0:n, :] + aggp_ref[1, 0:n, :] + y_ref[...]
        h = dinv_ref[...] * agg + b_ref[...]
        mean = jnp.mean(h, axis=0, keepdims=True)
        cen = h - mean
        var = jnp.mean(cen * cen, axis=0, keepdims=True)
        out_ref[...] = g_ref[...] * cen * lax.rsqrt(var + _EPS) + be_ref[...]

    return pl.pallas_call(
        body,
        out_shape=jax.ShapeDtypeStruct((n, d), jnp.float32),
    )


def kernel(x, edge_index, W1, b1, gamma1, beta1, alpha, W2, b2, gamma2,
           beta2):
    n, d_in = x.shape
    d_h = W1.shape[1]
    d_out = W2.shape[1]
    e = edge_index.shape[1]
    grain = NW * BC * CHUNK
    e_pad = ((e + grain - 1) // grain) * grain
    cpt = e_pad // (NW * CHUNK)
    n_pad = ((n + 8 + NW * 32 - 1) // (NW * 32)) * (NW * 32)

    pad = e_pad - e
    src_flat = jnp.concatenate(
        [edge_index[0], jnp.zeros((pad,), edge_index.dtype)])
    dst_flat = jnp.concatenate(
        [edge_index[1], jnp.full((pad,), n, edge_index.dtype)])
    src3 = src_flat.reshape(NW, cpt, CHUNK)
    dst3 = dst_flat.reshape(NW, cpt, CHUNK)

    b1r = b1.reshape(1, d_h)
    g1r = gamma1.reshape(1, d_h)
    be1r = beta1.reshape(1, d_h)
    ar = alpha.reshape(1, 1)
    b2r = b2.reshape(1, d_out)
    g2r = gamma2.reshape(1, d_out)
    be2r = beta2.reshape(1, d_out)

    degp = _deg_kernel(n_pad, cpt)(dst3)
    y1, dinv = _tc_first(n, d_h)(x, W1, degp)
    agg1 = _agg_kernel(n, n_pad, cpt, d_h)(y1, src3, dst3)
    y2 = _tc_mid(n, d_h)(agg1, y1, dinv, b1r, g1r, be1r, ar, W2)
    agg2 = _agg_kernel(n, n_pad, cpt, d_out)(y2, src3, dst3)
    out = _tc_last(n, d_out)(agg2, y2, dinv, b2r, g2r, be2r)
    return out
